# Initial kernel scaffold; baseline (speedup 1.0000x reference)
import jax
import jax.numpy as jnp
from jax.experimental import pallas as pl


def kernel(x, edge_index, ggnn_weight, W_ih, W_hh, b_ih, b_hh, W_out, b_out):
    raise NotImplementedError("write your pallas kernel here")


# speedup vs baseline: 4.5311x; 4.5311x over previous
import functools

import jax
import jax.numpy as jnp
from jax import lax
from jax.experimental import pallas as pl
from jax.experimental.pallas import tpu as pltpu
from jax.experimental.pallas import tpu_sc as plsc

N = 10000
E = 320000
D = 128
NC = 2
NS = 16
NW = NC * NS
EDGES_PER_W = E // NW
CHUNK = 80
N_CHUNKS = EDGES_PER_W // CHUNK
NPAD = 10112
ROWS_PER_TILE = NPAD // NS


def _seg_sum_body(h_hbm, src_hbm, dst_hbm, zero_hbm, s0_hbm, s1_hbm,
                  acc_shared, src_v, dst_v, rows_v, sem):
    c = lax.axis_index("c")
    s = lax.axis_index("s")
    wid = s * NC + c

    pltpu.sync_copy(zero_hbm.at[pl.ds(s * ROWS_PER_TILE, ROWS_PER_TILE)],
                    acc_shared.at[pl.ds(s * ROWS_PER_TILE, ROWS_PER_TILE)])
    plsc.subcore_barrier()

    base = wid * EDGES_PER_W

    def chunk_step(i, carry):
        off = base + i * CHUNK
        pltpu.sync_copy(src_hbm.at[pl.ds(off, CHUNK)], src_v)
        pltpu.sync_copy(dst_hbm.at[pl.ds(off, CHUNK)], dst_v)
        pltpu.async_copy(h_hbm.at[src_v], rows_v, sem).wait()
        pltpu.sync_copy(rows_v, acc_shared.at[dst_v], add=True)
        return carry

    lax.fori_loop(0, N_CHUNKS, chunk_step, 0)
    plsc.subcore_barrier()

    rows = acc_shared.at[pl.ds(s * ROWS_PER_TILE, ROWS_PER_TILE)]

    @pl.when(c == 0)
    def _():
        pltpu.sync_copy(rows, s0_hbm.at[pl.ds(s * ROWS_PER_TILE, ROWS_PER_TILE)])

    @pl.when(c == 1)
    def _():
        pltpu.sync_copy(rows, s1_hbm.at[pl.ds(s * ROWS_PER_TILE, ROWS_PER_TILE)])


@jax.jit
def _seg_sum_sc(h, src, dst, zero):
    mesh = plsc.VectorSubcoreMesh(core_axis_name="c", subcore_axis_name="s")
    fn = pl.kernel(
        _seg_sum_body,
        mesh=mesh,
        out_type=(jax.ShapeDtypeStruct((NPAD, D), jnp.float32),
                  jax.ShapeDtypeStruct((NPAD, D), jnp.float32)),
        scratch_types=[
            pltpu.VMEM_SHARED((NPAD, D), jnp.float32),
            pltpu.VMEM((CHUNK,), jnp.int32),
            pltpu.VMEM((CHUNK,), jnp.int32),
            pltpu.VMEM((CHUNK, D), jnp.float32),
            pltpu.SemaphoreType.DMA,
        ],
    )
    return fn(h, src, dst, zero)


def _gru_tc_body(s0, s1, h, wg, wih, whh, bih, bhh, out):
    agg = jnp.dot(s0[...] + s1[...], wg[...],
                  preferred_element_type=jnp.float32)
    gi = jnp.dot(agg, wih[...], preferred_element_type=jnp.float32) + bih[...]
    gh = jnp.dot(h[...], whh[...], preferred_element_type=jnp.float32) + bhh[...]
    i_r, i_z, i_n = gi[:, :D], gi[:, D:2 * D], gi[:, 2 * D:]
    h_r, h_z, h_n = gh[:, :D], gh[:, D:2 * D], gh[:, 2 * D:]
    r = jax.nn.sigmoid(i_r + h_r)
    z = jax.nn.sigmoid(i_z + h_z)
    n = jnp.tanh(i_n + r * h_n)
    out[...] = (1.0 - z) * n + z * h[...]


BLK = 2000


def _gru_tc(s0, s1, h, wg, wihT, whhT, bih2, bhh2):
    grid = (N // BLK,)
    row = lambda i: (i, 0)
    fix = lambda i: (0, 0)
    return pl.pallas_call(
        _gru_tc_body,
        grid=grid,
        in_specs=[
            pl.BlockSpec((BLK, D), row),
            pl.BlockSpec((BLK, D), row),
            pl.BlockSpec((BLK, D), row),
            pl.BlockSpec((D, D), fix),
            pl.BlockSpec((D, 3 * D), fix),
            pl.BlockSpec((D, 3 * D), fix),
            pl.BlockSpec((1, 3 * D), fix),
            pl.BlockSpec((1, 3 * D), fix),
        ],
        out_specs=pl.BlockSpec((BLK, D), row),
        out_shape=jax.ShapeDtypeStruct((N, D), jnp.float32),
    )(s0, s1, h, wg, wihT, whhT, bih2, bhh2)


def _mlp_tc_body(h, x, wh, wx, b, out):
    acc = jnp.dot(h[...], wh[...], preferred_element_type=jnp.float32)
    acc += jnp.dot(x[...], wx[...], preferred_element_type=jnp.float32)
    out[...] = jnp.maximum(acc + b[...], 0.0)


def _mlp_tc(h, x, whT, wxT, b2):
    grid = (N // BLK,)
    row = lambda i: (i, 0)
    fix = lambda i: (0, 0)
    return pl.pallas_call(
        _mlp_tc_body,
        grid=grid,
        in_specs=[
            pl.BlockSpec((BLK, D), row),
            pl.BlockSpec((BLK, D), row),
            pl.BlockSpec((D, D), fix),
            pl.BlockSpec((D, D), fix),
            pl.BlockSpec((1, D), fix),
        ],
        out_specs=pl.BlockSpec((BLK, D), row),
        out_shape=jax.ShapeDtypeStruct((N, D), jnp.float32),
    )(h, x, whT, wxT, b2)


def kernel(x, edge_index, ggnn_weight, W_ih, W_hh, b_ih, b_hh, W_out, b_out):
    src = edge_index[0]
    dst = edge_index[1]
    zero = jnp.zeros((NPAD, D), jnp.float32)
    wihT = W_ih.T
    whhT = W_hh.T
    bih2 = b_ih.reshape(1, 3 * D)
    bhh2 = b_hh.reshape(1, 3 * D)
    whT = W_out[:, :D].T
    wxT = W_out[:, D:].T
    b2 = b_out.reshape(1, -1)

    h = x
    for i in range(3):
        h_pad = jnp.pad(h, ((0, NPAD - N), (0, 0)))
        s0, s1 = _seg_sum_sc(h_pad, src, dst, zero)
        h = _gru_tc(s0, s1, h, ggnn_weight[i], wihT, whhT, bih2, bhh2)
    return _mlp_tc(h, x, whT, wxT, b2)
